# baseline (speedup 1.0000x reference)
import jax
import jax.numpy as jnp
from jax.experimental import pallas as pl

_K = 19264
_BM = 2048
_BK = 1024
_NSTEPS = 19
_NCHUNK = 2
_BC = _BK // _NCHUNK


def _fused_kernel(xt_ref, emb_ref, o_ref):
    j = pl.program_id(1)

    @pl.when(j == 0)
    def _init():
        o_ref[...] = jnp.zeros_like(o_ref)

    is_tail = j == _NSTEPS - 1
    valid = _K - (_NSTEPS - 1) * _BK

    def _load(c, lo, hi):
        xt = xt_ref[lo:hi, :]
        e = emb_ref[lo:hi, :]

        def _masked():
            row = jax.lax.broadcasted_iota(jnp.int32, xt.shape, 0) + lo
            erow = jax.lax.broadcasted_iota(jnp.int32, e.shape, 0) + lo
            return (
                jnp.where(row < valid, xt, 0.0),
                jnp.where(erow < valid, e, 0.0),
            )

        return jax.lax.cond(is_tail, _masked, lambda: (xt, e))

    acc = None
    for c in range(_NCHUNK):
        lo, hi = c * _BC, (c + 1) * _BC
        xt, e = _load(c, lo, hi)
        y = jnp.log(xt + 1.0).astype(jnp.bfloat16)
        d = jax.lax.dot_general(
            y,
            e.astype(jnp.bfloat16),
            (((0,), (0,)), ((), ())),
            preferred_element_type=jnp.float32,
        )
        acc = d if acc is None else acc + d
    o_ref[...] += acc


def kernel(x_seq, bb_gene_emb):
    m, k = x_seq.shape
    _, n = bb_gene_emb.shape
    xt = x_seq.T
    return pl.pallas_call(
        _fused_kernel,
        grid=(m // _BM, _NSTEPS),
        in_specs=[
            pl.BlockSpec((_BK, _BM), lambda i, j: (j, i)),
            pl.BlockSpec((_BK, n), lambda i, j: (j, 0)),
        ],
        out_specs=pl.BlockSpec((_BM, n), lambda i, j: (i, 0)),
        out_shape=jax.ShapeDtypeStruct((m, n), jnp.float32),
    )(xt, bb_gene_emb)


# speedup vs baseline: 1.9899x; 1.9899x over previous
import jax
import jax.numpy as jnp
from jax.experimental import pallas as pl

_K = 19264
_BM = 2048
_BK = 1024
_NSTEPS = 19
_NCHUNK = 2
_BC = _BK // _NCHUNK


def _fused_kernel(xt_ref, emb_ref, o_ref):
    j = pl.program_id(1)

    @pl.when(j == 0)
    def _init():
        o_ref[...] = jnp.zeros_like(o_ref)

    valid = _K - j * _BK

    acc = None
    for c in range(_NCHUNK):
        lo, hi = c * _BC, (c + 1) * _BC
        xt = xt_ref[lo:hi, :]
        row = jax.lax.broadcasted_iota(jnp.int32, xt.shape, 0) + lo
        xt = jnp.where(row < valid, xt, 0.0)
        y = jnp.log(xt + 1.0).astype(jnp.bfloat16)
        e = emb_ref[lo:hi, :]
        erow = jax.lax.broadcasted_iota(jnp.int32, e.shape, 0) + lo
        e = jnp.where(erow < valid, e, 0.0).astype(jnp.bfloat16)
        d = jax.lax.dot_general(
            y, e, (((0,), (0,)), ((), ())), preferred_element_type=jnp.float32
        )
        acc = d if acc is None else acc + d
    o_ref[...] += acc


def kernel(x_seq, bb_gene_emb):
    m, k = x_seq.shape
    _, n = bb_gene_emb.shape
    xt = x_seq.T
    return pl.pallas_call(
        _fused_kernel,
        grid=(m // _BM, _NSTEPS),
        in_specs=[
            pl.BlockSpec((_BK, _BM), lambda i, j: (j, i)),
            pl.BlockSpec((_BK, n), lambda i, j: (j, 0)),
        ],
        out_specs=pl.BlockSpec((_BM, n), lambda i, j: (i, 0)),
        out_shape=jax.ShapeDtypeStruct((m, n), jnp.float32),
    )(xt, bb_gene_emb)
